# baseline (speedup 1.0000x reference)
import functools

import jax
import jax.numpy as jnp
from jax import lax
from jax.experimental import pallas as pl
from jax.experimental.pallas import tpu as pltpu
from jax.experimental.pallas import tpu_sc as plsc

_N = 10000
_D = 128
_E = 320000

_NC = 2
_NS = 16
_NW = _NC * _NS
_EPW = _E // _NW
_CHUNK = 128
_NCHUNK = 80
_EPWP = _NCHUNK * _CHUNK
_NP = 10240
_RPT = _NP // _NS
_PADROW = _NP - _CHUNK

_mesh = plsc.VectorSubcoreMesh(core_axis_name="c", subcore_axis_name="s")


@functools.partial(
    pl.kernel,
    out_type=jax.ShapeDtypeStruct((_NC, _NP, _D), jnp.float32),
    mesh=_mesh,
    scratch_types=[
        pltpu.VMEM((_NCHUNK, _CHUNK), jnp.int32),
        pltpu.VMEM((_CHUNK,), jnp.int32),
        pltpu.VMEM((_CHUNK,), jnp.int32),
        pltpu.VMEM((_CHUNK,), jnp.int32),
        pltpu.VMEM((_CHUNK,), jnp.int32),
        pltpu.VMEM((_CHUNK, _D), jnp.float32),
        pltpu.VMEM((_CHUNK, _D), jnp.float32),
        pltpu.VMEM_SHARED((_NP, _D), jnp.float32),
        pltpu.SemaphoreType.DMA,
        pltpu.SemaphoreType.DMA,
    ])
def _sc_segsum(h_hbm, packed_hbm, z_hbm, sum_out,
               packed_v, srcA, srcB, dstA, dstB, rows0, rows1, acc_sh,
               sem0, sem1):
    c = lax.axis_index("c")
    s = lax.axis_index("s")
    wid = c * _NS + s

    pltpu.sync_copy(packed_hbm.at[wid], packed_v)
    r0 = s * _RPT
    pltpu.sync_copy(z_hbm.at[pl.ds(r0, _RPT)], acc_sh.at[pl.ds(r0, _RPT)])
    plsc.subcore_barrier()

    def unpack(i, src_v, dst_v):
        for k in range(_CHUNK // 16):
            p = packed_v[i, pl.ds(16 * k, 16)]
            src_v[pl.ds(16 * k, 16)] = lax.shift_right_logical(p, 16)
            dst_v[pl.ds(16 * k, 16)] = lax.bitwise_and(p, 0xFFFF)

    unpack(0, srcA, dstA)
    pltpu.async_copy(h_hbm.at[srcA], rows0, sem0)
    unpack(1, srcB, dstB)
    pltpu.async_copy(h_hbm.at[srcB], rows1, sem1)

    def body(j, carry):
        i = 2 * j
        pltpu.make_async_copy(h_hbm.at[srcA], rows0, sem0).wait()
        pltpu.sync_copy(rows0, acc_sh.at[dstA], add=True)
        unpack(i + 2, srcA, dstA)
        pltpu.async_copy(h_hbm.at[srcA], rows0, sem0)
        pltpu.make_async_copy(h_hbm.at[srcB], rows1, sem1).wait()
        pltpu.sync_copy(rows1, acc_sh.at[dstB], add=True)
        unpack(i + 3, srcB, dstB)
        pltpu.async_copy(h_hbm.at[srcB], rows1, sem1)
        return carry

    lax.fori_loop(0, _NCHUNK // 2 - 1, body, 0)
    pltpu.make_async_copy(h_hbm.at[srcA], rows0, sem0).wait()
    pltpu.sync_copy(rows0, acc_sh.at[dstA], add=True)
    pltpu.make_async_copy(h_hbm.at[srcB], rows1, sem1).wait()
    pltpu.sync_copy(rows1, acc_sh.at[dstB], add=True)
    plsc.subcore_barrier()

    pltpu.sync_copy(acc_sh.at[pl.ds(r0, _RPT)],
                    sum_out.at[c, pl.ds(r0, _RPT)])


@functools.partial(
    pl.kernel,
    out_type=jax.ShapeDtypeStruct((_NC, _NP, _D), jnp.float32),
    mesh=_mesh,
    scratch_types=[
        pltpu.VMEM((_NCHUNK, _CHUNK), jnp.int32),
        pltpu.VMEM((_CHUNK, _D), jnp.float32),
        pltpu.VMEM_SHARED((_NP, _D), jnp.float32),
    ])
def _sc_degree(dst_hbm, z_hbm, ones_hbm, cnt_out, dst_v, ones_v, cnt_sh):
    c = lax.axis_index("c")
    s = lax.axis_index("s")

    pltpu.sync_copy(dst_hbm.at[c * _NS + s], dst_v)
    r0 = s * _RPT
    pltpu.sync_copy(z_hbm.at[pl.ds(r0, _RPT)], cnt_sh.at[pl.ds(r0, _RPT)])
    pltpu.sync_copy(ones_hbm, ones_v)
    plsc.subcore_barrier()

    def body(i, carry):
        pltpu.sync_copy(ones_v, cnt_sh.at[dst_v.at[i]], add=True)
        return carry

    lax.fori_loop(0, _NCHUNK, body, 0)
    plsc.subcore_barrier()

    pltpu.sync_copy(cnt_sh.at[pl.ds(r0, _RPT)],
                    cnt_out.at[c, pl.ds(r0, _RPT)])


_BR = 400


def _tc_layer(h, sum2, cnt2, WlT, WrT, bl2d):
    def body(h_ref, s_ref, c_ref, wl_ref, wr_ref, b_ref, o_ref):
        ssum = s_ref[0] + s_ref[1]
        cnt = c_ref[0][:, 0:1] + c_ref[1][:, 0:1]
        mean = ssum / jnp.maximum(cnt, 1.0)
        acc = jax.lax.dot(mean, wl_ref[...],
                          precision=jax.lax.Precision.HIGHEST,
                          preferred_element_type=jnp.float32)
        acc = acc + jax.lax.dot(h_ref[...], wr_ref[...],
                                precision=jax.lax.Precision.HIGHEST,
                                preferred_element_type=jnp.float32)
        o_ref[...] = jnp.maximum(acc + b_ref[...], 0.0)

    return pl.pallas_call(
        body,
        grid=(_N // _BR,),
        in_specs=[
            pl.BlockSpec((_BR, _D), lambda i: (i, 0)),
            pl.BlockSpec((_NC, _BR, _D), lambda i: (0, i, 0)),
            pl.BlockSpec((_NC, _BR, _D), lambda i: (0, i, 0)),
            pl.BlockSpec((_D, _D), lambda i: (0, 0)),
            pl.BlockSpec((_D, _D), lambda i: (0, 0)),
            pl.BlockSpec((1, _D), lambda i: (0, 0)),
        ],
        out_specs=pl.BlockSpec((_BR, _D), lambda i: (i, 0)),
        out_shape=jax.ShapeDtypeStruct((_N, _D), jnp.float32),
    )(h, sum2, cnt2, WlT, WrT, bl2d)


def kernel(x, edge_index, Wl1, bl1, Wr1, Wl2, bl2, Wr2, Wl3, bl3, Wr3):
    pad = _EPWP - _EPW
    srcw = edge_index[0].reshape(_NW, _EPW)
    dstw = edge_index[1].reshape(_NW, _EPW)
    packedw = jnp.left_shift(srcw, 16) | dstw
    packed3 = jnp.pad(packedw, ((0, 0), (0, pad)),
                      constant_values=_PADROW).reshape(_NW, _NCHUNK, _CHUNK)
    dst3 = jnp.pad(dstw, ((0, 0), (0, pad)),
                   constant_values=_PADROW).reshape(_NW, _NCHUNK, _CHUNK)
    zeros = jnp.zeros((_NP, _D), jnp.float32)
    ones = jnp.ones((_CHUNK, _D), jnp.float32)

    h = x
    cnt2 = _sc_degree(dst3, zeros, ones)
    for Wl, bl, Wr in [(Wl1, bl1, Wr1), (Wl2, bl2, Wr2), (Wl3, bl3, Wr3)]:
        sum2 = _sc_segsum(h, packed3, zeros)
        h = _tc_layer(h, sum2[:, :_N], cnt2[:, :_N], Wl.T, Wr.T,
                      bl.reshape(1, _D))
    return h
